# Initial kernel scaffold; baseline (speedup 1.0000x reference)
import jax
import jax.numpy as jnp
from jax.experimental import pallas as pl


def kernel(boxes, scores):
    raise NotImplementedError("write your pallas kernel here")


# speedup vs baseline: 15.2283x; 15.2283x over previous
import jax
import jax.numpy as jnp
from jax import lax
from jax.experimental import pallas as pl
from jax.experimental.pallas import tpu as pltpu

TOPK = 3000
NPAD = 3072
B = 128
M = NPAD // B
IOU_T = 0.6


def _nms_body(b_ref, bTf_ref, bTb_ref, sT_ref, out_ref, keep_ref, mat_ref):
    i = pl.program_id(0)

    @pl.when(i == 0)
    def _init():
        keep_ref[...] = jnp.ones((M, B), jnp.float32)

    bb = b_ref[...]
    rx1, ry1, rx2, ry2 = bb[:, 0:1], bb[:, 1:2], bb[:, 2:3], bb[:, 3:4]
    rarea = (rx2 - rx1) * (ry2 - ry1)

    cb = bTb_ref[...]
    cx1, cy1, cx2, cy2 = cb[0:1, :], cb[1:2, :], cb[2:3, :], cb[3:4, :]
    carea = (cx2 - cx1) * (cy2 - cy1)

    xx1 = jnp.maximum(rx1, cx1)
    yy1 = jnp.maximum(ry1, cy1)
    xx2 = jnp.minimum(rx2, cx2)
    yy2 = jnp.minimum(ry2, cy2)
    iw = jnp.maximum(xx2 - xx1, 0.0)
    ih = jnp.maximum(yy2 - yy1, 0.0)
    inter = iw * ih
    union = rarea + carea - inter
    mat_ref[...] = inter / (union + 1e-9)

    col = lax.broadcasted_iota(jnp.int32, (1, B), 1)
    kb0 = keep_ref[pl.ds(i, 1), :]

    def step(k, kb):
        row = mat_ref[pl.ds(k, 1), :]
        kbk = jnp.sum(jnp.where(col == k, kb, 0.0))
        sup = (row > IOU_T) & (col > k) & (kbk > 0.0)
        return jnp.where(sup, 0.0, kb)

    kb = lax.fori_loop(0, B, step, kb0)
    keep_ref[pl.ds(i, 1), :] = kb

    ab = bTf_ref[...]
    ax1, ay1, ax2, ay2 = ab[0:1, :], ab[1:2, :], ab[2:3, :], ab[3:4, :]
    aarea = (ax2 - ax1) * (ay2 - ay1)
    xx1a = jnp.maximum(rx1, ax1)
    yy1a = jnp.maximum(ry1, ay1)
    xx2a = jnp.minimum(rx2, ax2)
    yy2a = jnp.minimum(ry2, ay2)
    iwa = jnp.maximum(xx2a - xx1a, 0.0)
    iha = jnp.maximum(yy2a - yy1a, 0.0)
    intera = iwa * iha
    uniona = rarea + aarea - intera
    ind = ((intera / (uniona + 1e-9)) > IOU_T).astype(jnp.float32)
    supc = jnp.dot(kb, ind, preferred_element_type=jnp.float32)
    sup2 = supc.reshape(M, B) > 0.0
    rows = lax.broadcasted_iota(jnp.int32, (M, B), 0)
    keep_ref[...] = jnp.where((rows > i) & sup2, 0.0, keep_ref[...])

    out_ref[0:4, :] = cb * kb
    out_ref[4:5, :] = sT_ref[...] * kb


def kernel(boxes, scores):
    top_scores, idx = lax.top_k(scores, TOPK)
    b = jnp.take(boxes, idx, axis=0)
    bp = jnp.concatenate([b, jnp.zeros((NPAD - TOPK, 4), b.dtype)], axis=0)
    sp = jnp.concatenate(
        [top_scores, jnp.zeros((NPAD - TOPK,), top_scores.dtype)]
    )
    bT = bp.T
    sT = sp[None, :]
    outT = pl.pallas_call(
        _nms_body,
        grid=(M,),
        in_specs=[
            pl.BlockSpec((B, 4), lambda i: (i, 0)),
            pl.BlockSpec((4, NPAD), lambda i: (0, 0)),
            pl.BlockSpec((4, B), lambda i: (0, i)),
            pl.BlockSpec((1, B), lambda i: (0, i)),
        ],
        out_specs=pl.BlockSpec((5, B), lambda i: (0, i)),
        out_shape=jax.ShapeDtypeStruct((5, NPAD), jnp.float32),
        scratch_shapes=[
            pltpu.VMEM((M, B), jnp.float32),
            pltpu.VMEM((B, B), jnp.float32),
        ],
    )(bp, bT, bT, sT)
    return outT.T[:TOPK]


# speedup vs baseline: 81.9666x; 5.3825x over previous
import jax
import jax.numpy as jnp
from jax import lax
from jax.experimental import pallas as pl
from jax.experimental.pallas import tpu as pltpu

TOPK = 3000
NPAD = 3072
B = 128
M = NPAD // B
IOU_T = 0.6


def _nms_body(b_ref, bTf_ref, bTb_ref, sT_ref, out_ref, keep_ref):
    i = pl.program_id(0)

    @pl.when(i == 0)
    def _init():
        keep_ref[...] = jnp.ones((M, B), jnp.float32)

    bb = b_ref[...]
    rx1, ry1, rx2, ry2 = bb[:, 0:1], bb[:, 1:2], bb[:, 2:3], bb[:, 3:4]
    rarea = (rx2 - rx1) * (ry2 - ry1)

    cb = bTb_ref[...]
    cx1, cy1, cx2, cy2 = cb[0:1, :], cb[1:2, :], cb[2:3, :], cb[3:4, :]
    carea = (cx2 - cx1) * (cy2 - cy1)

    xx1 = jnp.maximum(rx1, cx1)
    yy1 = jnp.maximum(ry1, cy1)
    xx2 = jnp.minimum(rx2, cx2)
    yy2 = jnp.minimum(ry2, cy2)
    iw = jnp.maximum(xx2 - xx1, 0.0)
    ih = jnp.maximum(yy2 - yy1, 0.0)
    inter = iw * ih
    union = rarea + carea - inter
    iou = inter / (union + 1e-9)
    rowi = lax.broadcasted_iota(jnp.int32, (B, B), 0)
    coli = lax.broadcasted_iota(jnp.int32, (B, B), 1)
    sup_ut = ((iou > IOU_T) & (rowi < coli)).astype(jnp.bfloat16)

    kb0 = keep_ref[pl.ds(i, 1), :]

    def fstep(x):
        hit = jnp.dot(
            x.astype(jnp.bfloat16), sup_ut, preferred_element_type=jnp.float32
        )
        return jnp.where(hit > 0.0, 0.0, kb0)

    def cond(c):
        xprev, x = c
        return jnp.any(xprev != x)

    def body(c):
        _, x = c
        return (x, fstep(x))

    _, kb = lax.while_loop(cond, body, (kb0, fstep(kb0)))
    keep_ref[pl.ds(i, 1), :] = kb

    ab = bTf_ref[...]
    ax1, ay1, ax2, ay2 = ab[0:1, :], ab[1:2, :], ab[2:3, :], ab[3:4, :]
    aarea = (ax2 - ax1) * (ay2 - ay1)
    xx1a = jnp.maximum(rx1, ax1)
    yy1a = jnp.maximum(ry1, ay1)
    xx2a = jnp.minimum(rx2, ax2)
    yy2a = jnp.minimum(ry2, ay2)
    iwa = jnp.maximum(xx2a - xx1a, 0.0)
    iha = jnp.maximum(yy2a - yy1a, 0.0)
    intera = iwa * iha
    uniona = rarea + aarea - intera
    ind = ((intera / (uniona + 1e-9)) > IOU_T).astype(jnp.bfloat16)
    supc = jnp.dot(
        kb.astype(jnp.bfloat16), ind, preferred_element_type=jnp.float32
    )
    sup2 = supc.reshape(M, B) > 0.0
    rows = lax.broadcasted_iota(jnp.int32, (M, B), 0)
    keep_ref[...] = jnp.where((rows > i) & sup2, 0.0, keep_ref[...])

    out_ref[0:4, :] = cb * kb
    out_ref[4:5, :] = sT_ref[...] * kb


def kernel(boxes, scores):
    top_scores, idx = lax.top_k(scores, TOPK)
    b = jnp.take(boxes, idx, axis=0)
    bp = jnp.concatenate([b, jnp.zeros((NPAD - TOPK, 4), b.dtype)], axis=0)
    sp = jnp.concatenate(
        [top_scores, jnp.zeros((NPAD - TOPK,), top_scores.dtype)]
    )
    bT = bp.T
    sT = sp[None, :]
    outT = pl.pallas_call(
        _nms_body,
        grid=(M,),
        in_specs=[
            pl.BlockSpec((B, 4), lambda i: (i, 0)),
            pl.BlockSpec((4, NPAD), lambda i: (0, 0)),
            pl.BlockSpec((4, B), lambda i: (0, i)),
            pl.BlockSpec((1, B), lambda i: (0, i)),
        ],
        out_specs=pl.BlockSpec((5, B), lambda i: (0, i)),
        out_shape=jax.ShapeDtypeStruct((5, NPAD), jnp.float32),
        scratch_shapes=[
            pltpu.VMEM((M, B), jnp.float32),
        ],
    )(bp, bT, bT, sT)
    return outT.T[:TOPK]


# speedup vs baseline: 87.0832x; 1.0624x over previous
import jax
import jax.numpy as jnp
from jax import lax
from jax.experimental import pallas as pl
from jax.experimental.pallas import tpu as pltpu

TOPK = 3000
NPAD = 3072
B = 256
M = NPAD // B
IOU_T = 0.6


def _nms_body(b_ref, bTf_ref, bTb_ref, sT_ref, out_ref, keep_ref):
    i = pl.program_id(0)

    @pl.when(i == 0)
    def _init():
        keep_ref[...] = jnp.ones((M, B), jnp.float32)

    bb = b_ref[...]
    rx1, ry1, rx2, ry2 = bb[:, 0:1], bb[:, 1:2], bb[:, 2:3], bb[:, 3:4]
    rarea = (rx2 - rx1) * (ry2 - ry1)

    cb = bTb_ref[...]
    cx1, cy1, cx2, cy2 = cb[0:1, :], cb[1:2, :], cb[2:3, :], cb[3:4, :]
    carea = (cx2 - cx1) * (cy2 - cy1)

    xx1 = jnp.maximum(rx1, cx1)
    yy1 = jnp.maximum(ry1, cy1)
    xx2 = jnp.minimum(rx2, cx2)
    yy2 = jnp.minimum(ry2, cy2)
    iw = jnp.maximum(xx2 - xx1, 0.0)
    ih = jnp.maximum(yy2 - yy1, 0.0)
    inter = iw * ih
    union = rarea + carea - inter
    iou = inter / (union + 1e-9)
    rowi = lax.broadcasted_iota(jnp.int32, (B, B), 0)
    coli = lax.broadcasted_iota(jnp.int32, (B, B), 1)
    sup_ut = ((iou > IOU_T) & (rowi < coli)).astype(jnp.bfloat16)

    kb0 = keep_ref[pl.ds(i, 1), :]

    def fstep(x):
        hit = jnp.dot(
            x.astype(jnp.bfloat16), sup_ut, preferred_element_type=jnp.float32
        )
        return jnp.where(hit > 0.0, 0.0, kb0)

    def cond(c):
        xprev, x = c
        return jnp.any(xprev != x)

    def body(c):
        _, x = c
        return (x, fstep(x))

    _, kb = lax.while_loop(cond, body, (kb0, fstep(kb0)))
    keep_ref[pl.ds(i, 1), :] = kb

    ab = bTf_ref[...]
    ax1, ay1, ax2, ay2 = ab[0:1, :], ab[1:2, :], ab[2:3, :], ab[3:4, :]
    aarea = (ax2 - ax1) * (ay2 - ay1)
    xx1a = jnp.maximum(rx1, ax1)
    yy1a = jnp.maximum(ry1, ay1)
    xx2a = jnp.minimum(rx2, ax2)
    yy2a = jnp.minimum(ry2, ay2)
    iwa = jnp.maximum(xx2a - xx1a, 0.0)
    iha = jnp.maximum(yy2a - yy1a, 0.0)
    intera = iwa * iha
    uniona = rarea + aarea - intera
    ind = ((intera / (uniona + 1e-9)) > IOU_T).astype(jnp.bfloat16)
    supc = jnp.dot(
        kb.astype(jnp.bfloat16), ind, preferred_element_type=jnp.float32
    )
    sup2 = supc.reshape(M, B) > 0.0
    rows = lax.broadcasted_iota(jnp.int32, (M, B), 0)
    keep_ref[...] = jnp.where((rows > i) & sup2, 0.0, keep_ref[...])

    out_ref[0:4, :] = cb * kb
    out_ref[4:5, :] = sT_ref[...] * kb


def kernel(boxes, scores):
    top_scores, idx = lax.top_k(scores, TOPK)
    b = jnp.take(boxes, idx, axis=0)
    bp = jnp.concatenate([b, jnp.zeros((NPAD - TOPK, 4), b.dtype)], axis=0)
    sp = jnp.concatenate(
        [top_scores, jnp.zeros((NPAD - TOPK,), top_scores.dtype)]
    )
    bT = bp.T
    sT = sp[None, :]
    outT = pl.pallas_call(
        _nms_body,
        grid=(M,),
        in_specs=[
            pl.BlockSpec((B, 4), lambda i: (i, 0)),
            pl.BlockSpec((4, NPAD), lambda i: (0, 0)),
            pl.BlockSpec((4, B), lambda i: (0, i)),
            pl.BlockSpec((1, B), lambda i: (0, i)),
        ],
        out_specs=pl.BlockSpec((5, B), lambda i: (0, i)),
        out_shape=jax.ShapeDtypeStruct((5, NPAD), jnp.float32),
        scratch_shapes=[
            pltpu.VMEM((M, B), jnp.float32),
        ],
    )(bp, bT, bT, sT)
    return outT.T[:TOPK]


# speedup vs baseline: 91.2197x; 1.0475x over previous
import jax
import jax.numpy as jnp
from jax import lax
from jax.experimental import pallas as pl
from jax.experimental.pallas import tpu as pltpu

TOPK = 3000
NPAD = 3072
B = 512
M = NPAD // B
IOU_T = 0.6


def _nms_body(b_ref, bTf_ref, bTb_ref, sT_ref, out_ref, keep_ref):
    i = pl.program_id(0)

    @pl.when(i == 0)
    def _init():
        keep_ref[...] = jnp.ones((M, B), jnp.float32)

    bb = b_ref[...]
    rx1, ry1, rx2, ry2 = bb[:, 0:1], bb[:, 1:2], bb[:, 2:3], bb[:, 3:4]
    rarea = (rx2 - rx1) * (ry2 - ry1)

    cb = bTb_ref[...]
    cx1, cy1, cx2, cy2 = cb[0:1, :], cb[1:2, :], cb[2:3, :], cb[3:4, :]
    carea = (cx2 - cx1) * (cy2 - cy1)

    xx1 = jnp.maximum(rx1, cx1)
    yy1 = jnp.maximum(ry1, cy1)
    xx2 = jnp.minimum(rx2, cx2)
    yy2 = jnp.minimum(ry2, cy2)
    iw = jnp.maximum(xx2 - xx1, 0.0)
    ih = jnp.maximum(yy2 - yy1, 0.0)
    inter = iw * ih
    union = rarea + carea - inter
    iou = inter / (union + 1e-9)
    rowi = lax.broadcasted_iota(jnp.int32, (B, B), 0)
    coli = lax.broadcasted_iota(jnp.int32, (B, B), 1)
    sup_ut = ((iou > IOU_T) & (rowi < coli)).astype(jnp.bfloat16)

    kb0 = keep_ref[pl.ds(i, 1), :]

    def fstep(x):
        hit = jnp.dot(
            x.astype(jnp.bfloat16), sup_ut, preferred_element_type=jnp.float32
        )
        return jnp.where(hit > 0.0, 0.0, kb0)

    def cond(c):
        xprev, x = c
        return jnp.any(xprev != x)

    def body(c):
        _, x = c
        return (x, fstep(x))

    _, kb = lax.while_loop(cond, body, (kb0, fstep(kb0)))
    keep_ref[pl.ds(i, 1), :] = kb

    ab = bTf_ref[...]
    ax1, ay1, ax2, ay2 = ab[0:1, :], ab[1:2, :], ab[2:3, :], ab[3:4, :]
    aarea = (ax2 - ax1) * (ay2 - ay1)
    xx1a = jnp.maximum(rx1, ax1)
    yy1a = jnp.maximum(ry1, ay1)
    xx2a = jnp.minimum(rx2, ax2)
    yy2a = jnp.minimum(ry2, ay2)
    iwa = jnp.maximum(xx2a - xx1a, 0.0)
    iha = jnp.maximum(yy2a - yy1a, 0.0)
    intera = iwa * iha
    uniona = rarea + aarea - intera
    ind = ((intera / (uniona + 1e-9)) > IOU_T).astype(jnp.bfloat16)
    supc = jnp.dot(
        kb.astype(jnp.bfloat16), ind, preferred_element_type=jnp.float32
    )
    sup2 = supc.reshape(M, B) > 0.0
    rows = lax.broadcasted_iota(jnp.int32, (M, B), 0)
    keep_ref[...] = jnp.where((rows > i) & sup2, 0.0, keep_ref[...])

    out_ref[0:4, :] = cb * kb
    out_ref[4:5, :] = sT_ref[...] * kb


def kernel(boxes, scores):
    top_scores, idx = lax.top_k(scores, TOPK)
    b = jnp.take(boxes, idx, axis=0)
    bp = jnp.concatenate([b, jnp.zeros((NPAD - TOPK, 4), b.dtype)], axis=0)
    sp = jnp.concatenate(
        [top_scores, jnp.zeros((NPAD - TOPK,), top_scores.dtype)]
    )
    bT = bp.T
    sT = sp[None, :]
    outT = pl.pallas_call(
        _nms_body,
        grid=(M,),
        in_specs=[
            pl.BlockSpec((B, 4), lambda i: (i, 0)),
            pl.BlockSpec((4, NPAD), lambda i: (0, 0)),
            pl.BlockSpec((4, B), lambda i: (0, i)),
            pl.BlockSpec((1, B), lambda i: (0, i)),
        ],
        out_specs=pl.BlockSpec((5, B), lambda i: (0, i)),
        out_shape=jax.ShapeDtypeStruct((5, NPAD), jnp.float32),
        scratch_shapes=[
            pltpu.VMEM((M, B), jnp.float32),
        ],
    )(bp, bT, bT, sT)
    return outT.T[:TOPK]
